# baseline (speedup 1.0000x reference)
import functools

import jax
import jax.numpy as jnp
from jax import lax
from jax.experimental import pallas as pl
from jax.experimental.pallas import tpu as pltpu
from jax.experimental.pallas import tpu_sc as plsc

_VOCAB = 1000000
_DIM = 32
_SEQ = 200
_BATCH = 4096

_NC = 2
_NS = 16
_NW = _NC * _NS
_BPW = _BATCH // _NW
_CH = 10
_NCHUNK = _SEQ // _CH
_ROWS = _CH * _BPW
_SUB = 16
_NSUB = _BPW // _SUB


@functools.partial(
    pl.kernel,
    out_type=jax.ShapeDtypeStruct((_BATCH,), jnp.float32),
    mesh=plsc.VectorSubcoreMesh(core_axis_name="c", subcore_axis_name="s"),
    compiler_params=pltpu.CompilerParams(use_tc_tiling_on_sc=False),
    scratch_types=[
        pltpu.VMEM((_SEQ * _DIM,), jnp.float32),
        pltpu.VMEM((16,), jnp.float32),
        pltpu.VMEM((_ROWS,), jnp.int32),
        pltpu.VMEM((_ROWS,), jnp.int32),
        pltpu.VMEM((_ROWS, _DIM), jnp.float32),
        pltpu.VMEM((_ROWS, _DIM), jnp.float32),
        pltpu.VMEM((_BPW, _DIM), jnp.float32),
        pltpu.VMEM((_BPW,), jnp.float32),
        pltpu.VMEM((32,), jnp.float32),
        pltpu.SemaphoreType.DMA,
        pltpu.SemaphoreType.DMA,
    ],
)
def _emb_linear_sc(x_hbm, table_hbm, w_hbm, bias_hbm, out_hbm,
                   w_v, bias_v, idx_a, idx_b, rows_a, rows_b, acc_v, out_v,
                   pad_v, sem_a, sem_b):
    wid = lax.axis_index("s") * _NC + lax.axis_index("c")
    base_b = wid * _BPW

    pltpu.sync_copy(w_hbm, w_v)
    pltpu.sync_copy(bias_hbm, bias_v)
    bias_s = bias_v[...][0]

    lanes = lax.iota(jnp.int32, 16)
    zero16 = jnp.zeros((16,), jnp.float32)
    pad_v[pl.ds(16, 16)] = zero16

    def zacc(i, carry):
        acc_v[i, pl.ds(0, 16)] = zero16
        acc_v[i, pl.ds(16, 16)] = zero16
        return carry
    lax.fori_loop(0, _BPW, zacc, 0)

    def _fetch(c, idx_v, rows_v, sem):
        for j in range(_CH):
            pltpu.sync_copy(x_hbm.at[c * _CH + j, pl.ds(base_b, _BPW)],
                            idx_v.at[pl.ds(j * _BPW, _BPW)])
        pltpu.async_copy(table_hbm.at[idx_v], rows_v, sem)

    def _reduce_chunk(c, rows_v):
        def sub_body(sub, carry):
            b0 = sub * _SUB

            def j_body(j, acc):
                wlo = w_v[pl.ds((c * _CH + j) * _DIM, 16)]
                whi = w_v[pl.ds((c * _CH + j) * _DIM + 16, 16)]
                out_acc = []
                for bl in range(_SUB):
                    r = j * _BPW + b0 + bl
                    lo = rows_v[r, pl.ds(0, 16)]
                    hi = rows_v[r, pl.ds(16, 16)]
                    out_acc.append(acc[2 * bl] + lo * wlo)
                    out_acc.append(acc[2 * bl + 1] + hi * whi)
                return tuple(out_acc)

            init = []
            for bl in range(_SUB):
                init.append(acc_v[b0 + bl, pl.ds(0, 16)])
                init.append(acc_v[b0 + bl, pl.ds(16, 16)])
            acc = lax.fori_loop(0, _CH, j_body, tuple(init))
            for bl in range(_SUB):
                acc_v[b0 + bl, pl.ds(0, 16)] = acc[2 * bl]
                acc_v[b0 + bl, pl.ds(16, 16)] = acc[2 * bl + 1]
            return carry

        lax.fori_loop(0, _NSUB, sub_body, 0)

    _fetch(0, idx_a, rows_a, sem_a)

    def pair(p, carry):
        pltpu.make_async_copy(table_hbm.at[idx_a], rows_a, sem_a).wait()
        _fetch(2 * p + 1, idx_b, rows_b, sem_b)
        _reduce_chunk(2 * p, rows_a)
        pltpu.make_async_copy(table_hbm.at[idx_b], rows_b, sem_b).wait()
        _fetch((2 * p + 2) % _NCHUNK, idx_a, rows_a, sem_a)
        _reduce_chunk(2 * p + 1, rows_b)
        return carry

    lax.fori_loop(0, _NCHUNK // 2, pair, 0)
    pltpu.make_async_copy(table_hbm.at[idx_a], rows_a, sem_a).wait()

    def _hsum(c):
        v = c
        for sh in (8, 4, 2, 1):
            pad_v[pl.ds(0, 16)] = v
            v = pad_v[pl.ds(0, 16)] + pad_v[pl.ds(sh, 16)]
        return v[0]

    def group(g, carry):
        vec = jnp.full((16,), bias_s, jnp.float32)
        for bl in range(16):
            lo = acc_v[g * 16 + bl, pl.ds(0, 16)]
            hi = acc_v[g * 16 + bl, pl.ds(16, 16)]
            total = _hsum(lo + hi)
            vec = jnp.where(lanes == bl, total, vec)
        out_v[pl.ds(g * 16, 16)] = vec
        return carry

    lax.fori_loop(0, _BPW // 16, group, 0)
    pltpu.sync_copy(out_v, out_hbm.at[pl.ds(base_b, _BPW)])


@jax.jit
def kernel(x, table, W, b):
    w_flat = W.reshape(-1)
    b16 = jnp.broadcast_to(b, (16,))
    return _emb_linear_sc(x, table, w_flat, b16)


# speedup vs baseline: 1.1759x; 1.1759x over previous
import functools

import jax
import jax.numpy as jnp
from jax import lax
from jax.experimental import pallas as pl
from jax.experimental.pallas import tpu as pltpu
from jax.experimental.pallas import tpu_sc as plsc

_VOCAB = 1000000
_DIM = 32
_SEQ = 200
_BATCH = 4096

_NC = 2
_NS = 16
_NW = _NC * _NS
_BPW = _BATCH // _NW
_CH = 10
_NCHUNK = _SEQ // _CH
_ROWS = _CH * _BPW
_SUB = 16
_NSUB = _BPW // _SUB


_TR = _SEQ // 8
_TC = _BATCH // 128
_TILES = _TR * _TC
_TPW = _TILES // _NW


@functools.partial(
    pl.kernel,
    out_type=jax.ShapeDtypeStruct((_SEQ * _BATCH,), jnp.int32),
    mesh=plsc.VectorSubcoreMesh(core_axis_name="c", subcore_axis_name="s"),
    compiler_params=pltpu.CompilerParams(use_tc_tiling_on_sc=True),
    scratch_types=[
        pltpu.VMEM((_TPW, 8, 128), jnp.int32),
        pltpu.SemaphoreType.DMA,
        pltpu.SemaphoreType.DMA,
    ],
)
def _detile_x(x_hbm, out_hbm, tiles_v, sem_in, sem_out):
    wid = lax.axis_index("s") * _NC + lax.axis_index("c")
    t0 = wid * _TPW
    for i in range(_TPW):
        t = t0 + i
        r8 = (t // _TC) * 8
        c128 = (t % _TC) * 128
        pltpu.async_copy(x_hbm.at[pl.ds(r8, 8), pl.ds(c128, 128)],
                         tiles_v.at[i], sem_in)
    for i in range(_TPW):
        pltpu.make_async_copy(x_hbm.at[pl.ds(0, 8), pl.ds(0, 128)],
                              tiles_v.at[i], sem_in).wait()
    for i in range(_TPW):
        t = t0 + i
        r8 = (t // _TC) * 8
        c128 = (t % _TC) * 128
        for r in range(8):
            pltpu.async_copy(tiles_v.at[i, r],
                             out_hbm.at[pl.ds((r8 + r) * _BATCH + c128, 128)],
                             sem_out)
    for i in range(_TPW):
        for r in range(8):
            pltpu.make_async_copy(tiles_v.at[i, r],
                                  out_hbm.at[pl.ds(0, 128)], sem_out).wait()


@functools.partial(
    pl.kernel,
    out_type=jax.ShapeDtypeStruct((_BATCH,), jnp.float32),
    mesh=plsc.VectorSubcoreMesh(core_axis_name="c", subcore_axis_name="s"),
    compiler_params=pltpu.CompilerParams(use_tc_tiling_on_sc=False),
    scratch_types=[
        pltpu.VMEM((_SEQ * _DIM,), jnp.float32),
        pltpu.VMEM((16,), jnp.float32),
        pltpu.VMEM((_ROWS,), jnp.int32),
        pltpu.VMEM((_ROWS,), jnp.int32),
        pltpu.VMEM((_ROWS, _DIM), jnp.float32),
        pltpu.VMEM((_ROWS, _DIM), jnp.float32),
        pltpu.VMEM((_BPW, _DIM), jnp.float32),
        pltpu.VMEM((_BPW,), jnp.float32),
        pltpu.VMEM((32,), jnp.float32),
        pltpu.SemaphoreType.DMA,
        pltpu.SemaphoreType.DMA,
        pltpu.SemaphoreType.DMA,
        pltpu.SemaphoreType.DMA,
    ],
)
def _emb_linear_sc(x_hbm, table_hbm, w_hbm, bias_hbm, out_hbm,
                   w_v, bias_v, idx_a, idx_b, rows_a, rows_b, acc_v, out_v,
                   pad_v, sem_ra, sem_rb, sem_ia, sem_ib):
    wid = lax.axis_index("s") * _NC + lax.axis_index("c")
    base_b = wid * _BPW

    pltpu.sync_copy(w_hbm, w_v)
    pltpu.sync_copy(bias_hbm, bias_v)
    bias_s = bias_v[...][0]

    lanes = lax.iota(jnp.int32, 16)
    zero16 = jnp.zeros((16,), jnp.float32)
    pad_v[pl.ds(16, 16)] = zero16

    def zacc(i, carry):
        acc_v[i, pl.ds(0, 16)] = zero16
        acc_v[i, pl.ds(16, 16)] = zero16
        return carry
    lax.fori_loop(0, _BPW, zacc, 0)

    def _fire_idx(c, idx_v, sem):
        cm = c % _NCHUNK
        for j in range(_CH):
            pltpu.async_copy(
                x_hbm.at[pl.ds((cm * _CH + j) * _BATCH + base_b, _BPW)],
                idx_v.at[pl.ds(j * _BPW, _BPW)], sem)

    def _wait_idx(idx_v, sem):
        for j in range(_CH):
            pltpu.make_async_copy(x_hbm.at[pl.ds(base_b, _BPW)],
                                  idx_v.at[pl.ds(j * _BPW, _BPW)], sem).wait()

    def _fire_gather(idx_v, rows_v, sem):
        pltpu.async_copy(table_hbm.at[idx_v], rows_v, sem)

    def _wait_gather(idx_v, rows_v, sem):
        pltpu.make_async_copy(table_hbm.at[idx_v], rows_v, sem).wait()

    def _reduce_chunk(c, rows_v):
        def sub_body(sub, carry):
            b0 = sub * _SUB

            def j_body(j, acc):
                wlo = w_v[pl.ds((c * _CH + j) * _DIM, 16)]
                whi = w_v[pl.ds((c * _CH + j) * _DIM + 16, 16)]
                out_acc = []
                for bl in range(_SUB):
                    r = j * _BPW + b0 + bl
                    lo = rows_v[r, pl.ds(0, 16)]
                    hi = rows_v[r, pl.ds(16, 16)]
                    out_acc.append(acc[2 * bl] + lo * wlo)
                    out_acc.append(acc[2 * bl + 1] + hi * whi)
                return tuple(out_acc)

            init = []
            for bl in range(_SUB):
                init.append(acc_v[b0 + bl, pl.ds(0, 16)])
                init.append(acc_v[b0 + bl, pl.ds(16, 16)])
            acc = lax.fori_loop(0, _CH, j_body, tuple(init))
            for bl in range(_SUB):
                acc_v[b0 + bl, pl.ds(0, 16)] = acc[2 * bl]
                acc_v[b0 + bl, pl.ds(16, 16)] = acc[2 * bl + 1]
            return carry

        lax.fori_loop(0, _NSUB, sub_body, 0)

    _fire_idx(0, idx_a, sem_ia)
    _wait_idx(idx_a, sem_ia)
    _fire_gather(idx_a, rows_a, sem_ra)
    _fire_idx(1, idx_b, sem_ib)

    def pair(p, carry):
        _wait_idx(idx_b, sem_ib)
        _fire_gather(idx_b, rows_b, sem_rb)
        _wait_gather(idx_a, rows_a, sem_ra)
        _fire_idx(2 * p + 2, idx_a, sem_ia)
        _reduce_chunk(2 * p, rows_a)
        _wait_idx(idx_a, sem_ia)
        _fire_gather(idx_a, rows_a, sem_ra)
        _wait_gather(idx_b, rows_b, sem_rb)
        _fire_idx(2 * p + 3, idx_b, sem_ib)
        _reduce_chunk(2 * p + 1, rows_b)
        return carry

    lax.fori_loop(0, _NCHUNK // 2, pair, 0)
    _wait_gather(idx_a, rows_a, sem_ra)
    _wait_idx(idx_b, sem_ib)

    def _hsum(c):
        v = c
        for sh in (8, 4, 2, 1):
            pad_v[pl.ds(0, 16)] = v
            v = pad_v[pl.ds(0, 16)] + pad_v[pl.ds(sh, 16)]
        return v[0]

    def group(g, carry):
        vec = jnp.full((16,), bias_s, jnp.float32)
        for bl in range(16):
            lo = acc_v[g * 16 + bl, pl.ds(0, 16)]
            hi = acc_v[g * 16 + bl, pl.ds(16, 16)]
            total = _hsum(lo + hi)
            vec = jnp.where(lanes == bl, total, vec)
        out_v[pl.ds(g * 16, 16)] = vec
        return carry

    lax.fori_loop(0, _BPW // 16, group, 0)
    pltpu.sync_copy(out_v, out_hbm.at[pl.ds(base_b, _BPW)])


@jax.jit
def kernel(x, table, W, b):
    w_flat = W.reshape(-1)
    b16 = jnp.broadcast_to(b, (16,))
    x_flat = _detile_x(x)
    return _emb_linear_sc(x_flat, table, w_flat, b16)
